# Initial kernel scaffold; baseline (speedup 1.0000x reference)
import jax
import jax.numpy as jnp
from jax.experimental import pallas as pl


def kernel(input, edge_index, edge_weight, weight, bias):
    raise NotImplementedError("write your pallas kernel here")


# speedup vs baseline: 2.7185x; 2.7185x over previous
import functools

import jax
import jax.numpy as jnp
from jax import lax
from jax.experimental import pallas as pl
from jax.experimental.pallas import tpu as pltpu
from jax.experimental.pallas import tpu_sc as plsc

N_NODES = 10000
D = 128
E = 320000
NC = 2
NS = 16
L = 16

B = 128
NBATCH = 80
E_TILE = B * NBATCH
E_PAD = E_TILE * NC * NS

ACC_ROWS = 10240
ROWS_PER_TILE = ACC_ROWS // NS


def _agg_body(x_hbm, src_hbm, dst_hbm, ew_hbm, out_hbm,
              src_v, dst_v, ew_v, rows_v, acc_sh, sem):
    c = lax.axis_index("c")
    s = lax.axis_index("s")
    wid = c * NS + s

    if True:
        zeros16 = jnp.zeros((L,), jnp.float32)

        def zrow(r, carry):
            for f in range(D // L):
                rows_v[r, pl.ds(f * L, L)] = zeros16
            return carry

        lax.fori_loop(0, B, zrow, 0)
        for k in range(ROWS_PER_TILE // B):
            pltpu.sync_copy(rows_v, acc_sh.at[pl.ds(s * ROWS_PER_TILE + k * B, B)])
        plsc.subcore_barrier()

        pltpu.sync_copy(src_hbm.at[wid], src_v)
        pltpu.sync_copy(dst_hbm.at[wid], dst_v)
        pltpu.sync_copy(ew_hbm.at[wid], ew_v)

        def body(b, carry):
            pltpu.async_copy(x_hbm.at[src_v.at[b]], rows_v, sem).wait()

            def mul_group(g, carry2):
                w_vreg = ew_v[b, pl.ds(g * L, L)]
                for j in range(L):
                    wb = w_vreg.at[jnp.full((L,), j, jnp.int32)].get(
                        mode="promise_in_bounds")
                    e = g * L + j
                    for f in range(D // L):
                        rows_v[e, pl.ds(f * L, L)] = (
                            rows_v[e, pl.ds(f * L, L)] * wb)
                return carry2

            lax.fori_loop(0, B // L, mul_group, 0)
            pltpu.sync_copy(rows_v, acc_sh.at[dst_v.at[b]], add=True)
            return carry

        lax.fori_loop(0, NBATCH, body, 0)
        plsc.subcore_barrier()

        @pl.when(s < NS - 1)
        def _full():
            pltpu.sync_copy(acc_sh.at[pl.ds(s * ROWS_PER_TILE, ROWS_PER_TILE)],
                            out_hbm.at[c, pl.ds(s * ROWS_PER_TILE, ROWS_PER_TILE)])

        @pl.when(s == NS - 1)
        def _tail():
            last = (NS - 1) * ROWS_PER_TILE
            pltpu.sync_copy(acc_sh.at[pl.ds(last, N_NODES - last)],
                            out_hbm.at[c, pl.ds(last, N_NODES - last)])

_agg = functools.partial(
    pl.kernel,
    out_type=jax.ShapeDtypeStruct((NC, N_NODES, D), jnp.float32),
    mesh=plsc.VectorSubcoreMesh(core_axis_name="c", subcore_axis_name="s"),
    scratch_types=[
        pltpu.VMEM((NBATCH, B), jnp.int32),
        pltpu.VMEM((NBATCH, B), jnp.int32),
        pltpu.VMEM((NBATCH, B), jnp.float32),
        pltpu.VMEM((B, D), jnp.float32),
        pltpu.VMEM_SHARED((ACC_ROWS, D), jnp.float32),
        pltpu.SemaphoreType.DMA,
    ],
)(_agg_body)


def _mm_body(p_ref, w_ref, b_ref, o_ref):
    acc = p_ref[0] + p_ref[1]
    o_ref[...] = (
        jnp.dot(acc, w_ref[...], preferred_element_type=jnp.float32)
        + b_ref[...]
    )


_MM_BLK = 1000

_mm = pl.pallas_call(
    _mm_body,
    grid=(N_NODES // _MM_BLK,),
    in_specs=[
        pl.BlockSpec((NC, _MM_BLK, D), lambda i: (0, i, 0)),
        pl.BlockSpec((D, D), lambda i: (0, 0)),
        pl.BlockSpec((1, D), lambda i: (0, 0)),
    ],
    out_specs=pl.BlockSpec((_MM_BLK, D), lambda i: (i, 0)),
    out_shape=jax.ShapeDtypeStruct((N_NODES, D), jnp.float32),
)


def kernel(input, edge_index, edge_weight, weight, bias):
    dst = edge_index[0]
    src = edge_index[1]
    pad = E_PAD - E
    src_p = jnp.pad(src, (0, pad)).reshape(NC * NS, NBATCH, B)
    dst_p = jnp.pad(dst, (0, pad)).reshape(NC * NS, NBATCH, B)
    ew_p = jnp.pad(edge_weight, (0, pad)).reshape(NC * NS, NBATCH, B)
    partials = _agg(input, src_p, dst_p, ew_p)
    return _mm(partials, weight, bias.reshape(1, D))
